# Initial kernel scaffold; baseline (speedup 1.0000x reference)
import jax
import jax.numpy as jnp
from jax.experimental import pallas as pl
from jax.experimental.pallas import tpu as pltpu

C_IN, H_IN, W_IN = 3, 32, 32
K = 5
C1, H1, W1 = 6, 28, 28
PH1 = 14
C2, H2, W2 = 16, 10, 10
PH2, PW2 = 5, 5
FC1, FC2, NCLS = 120, 84, 10

L1 = W1 * C1
L1P = L1 - C1
L2 = W2 * C2
L2P = L2 - C2
KC1 = C_IN * K * W_IN
KC2 = K * L1P
KF1 = PH2 * L2P


def _fused_kernel(x_ref, t1_ref, b1_ref, t2_ref, b2_ref, f1_ref, bf1_ref,
                  w2_ref, bf2_ref, w3_ref, b3_ref, o_ref):
    f32 = jnp.float32
    B = x_ref.shape[0]
    xb = x_ref[...]

    lhs1 = jnp.concatenate(
        [xb[:, ci * H_IN + kh: ci * H_IN + kh + H1, :]
         for ci in range(C_IN) for kh in range(K)], axis=-1)
    c1 = jnp.dot(lhs1.reshape(B * H1, KC1), t1_ref[...],
                 preferred_element_type=f32) + b1_ref[...]
    c1 = jnp.maximum(c1, 0.0).reshape(B, H1, L1)

    m1 = jnp.maximum(c1[:, :, 0:L1P], c1[:, :, C1:L1])
    p1 = jnp.maximum(m1[:, 0:H1:2, :], m1[:, 1:H1:2, :])

    lhs2 = jnp.concatenate(
        [p1[:, kh:kh + H2, :] for kh in range(K)], axis=-1)
    c2 = jnp.dot(lhs2.reshape(B * H2, KC2), t2_ref[...],
                 preferred_element_type=f32) + b2_ref[...]
    c2 = jnp.maximum(c2, 0.0).reshape(B, H2, L2)

    m2 = jnp.maximum(c2[:, :, 0:L2P], c2[:, :, C2:L2])
    p2 = jnp.maximum(m2[:, 0:H2:2, :], m2[:, 1:H2:2, :])

    lhs3 = jnp.concatenate([p2[:, r, :] for r in range(PH2)], axis=-1)
    y1 = jnp.maximum(jnp.dot(lhs3, f1_ref[...], preferred_element_type=f32)
                     + bf1_ref[...], 0.0)
    y2 = jnp.maximum(jnp.dot(y1, w2_ref[...], preferred_element_type=f32)
                     + bf2_ref[...], 0.0)
    o_ref[...] = (jnp.dot(y2, w3_ref[...], preferred_element_type=f32)
                  + b3_ref[...])


def kernel(x, t1, b1, t2, b2, f1, bf1, w2, bf2, w3, b3):
    n = x.shape[0]
    B = next(b for b in (128, 64, 32, 16, 8, 4, 2, 1) if n % b == 0)

    xr = x.reshape(n, C_IN * H_IN, W_IN).astype(jnp.float32)
    T1 = t1.transpose(1, 0, 2, 3).reshape(KC1, L1)
    T2 = t2.reshape(KC2, L2)
    F1 = f1.reshape(KF1, FC1)

    def full(shape):
        return pl.BlockSpec(shape, lambda i: (0,) * len(shape))

    out = pl.pallas_call(
        _fused_kernel,
        out_shape=jax.ShapeDtypeStruct((n, NCLS), jnp.float32),
        grid=(n // B,),
        in_specs=[
            pl.BlockSpec((B, C_IN * H_IN, W_IN), lambda i: (i, 0, 0)),
            full((KC1, L1)),
            full((1, L1)),
            full((KC2, L2)),
            full((1, L2)),
            full((KF1, FC1)),
            full((1, FC1)),
            full((FC1, FC2)),
            full((1, FC2)),
            full((FC2, NCLS)),
            full((1, NCLS)),
        ],
        out_specs=pl.BlockSpec((B, NCLS), lambda i: (i, 0)),
        compiler_params=pltpu.CompilerParams(
            dimension_semantics=("parallel",)),
    )(xr, T1, b1, T2, b2, F1, bf1, w2, bf2, w3, b3)
    return out


# speedup vs baseline: 9.5929x; 9.5929x over previous
import jax
import jax.numpy as jnp
from jax.experimental import pallas as pl
from jax.experimental.pallas import tpu as pltpu

C_IN, H_IN, W_IN = 3, 32, 32
K = 5
C1, H1, W1 = 6, 28, 28
PH1 = 14
C2, H2, W2 = 16, 10, 10
PH2, PW2 = 5, 5
FC1, FC2, NCLS = 120, 84, 10

L1 = W1 * C1
L1P = L1 - C1
L2 = W2 * C2
L2P = L2 - C2
KC1 = C_IN * K * W_IN
KC2 = K * L1P
KF1 = PH2 * L2P


def _fused_kernel(x_ref, t1_ref, b1_ref, t2_ref, b2_ref, f1_ref, bf1_ref,
                  w2_ref, bf2_ref, w3_ref, b3_ref, o_ref):
    f32 = jnp.float32
    bf16 = jnp.bfloat16
    B = x_ref.shape[0]
    xb = x_ref[...].astype(bf16)

    xcat = jnp.concatenate(
        [xb[:, ci * H_IN:(ci + 1) * H_IN, :] for ci in range(C_IN)],
        axis=-1)
    lhs1 = jnp.concatenate(
        [xcat[:, kh:kh + H1, :] for kh in range(K)], axis=-1)
    c1 = jnp.dot(lhs1.reshape(B * H1, KC1), t1_ref[...],
                 preferred_element_type=f32) + b1_ref[...]
    c1 = jnp.maximum(c1, 0.0).reshape(B, H1, L1).astype(bf16)

    m1 = jnp.maximum(c1[:, :, 0:L1P], c1[:, :, C1:L1])
    m1 = m1.reshape(B, PH1, 2, L1P)
    p1 = jnp.maximum(m1[:, :, 0, :], m1[:, :, 1, :])

    lhs2 = jnp.concatenate(
        [p1[:, kh:kh + H2, :] for kh in range(K)], axis=-1)
    c2 = jnp.dot(lhs2.reshape(B * H2, KC2), t2_ref[...],
                 preferred_element_type=f32) + b2_ref[...]
    c2 = jnp.maximum(c2, 0.0).reshape(B, H2, L2).astype(bf16)

    m2 = jnp.maximum(c2[:, :, 0:L2P], c2[:, :, C2:L2])
    m2 = m2.reshape(B, PH2, 2, L2P)
    p2 = jnp.maximum(m2[:, :, 0, :], m2[:, :, 1, :])

    lhs3 = jnp.concatenate([p2[:, r, :] for r in range(PH2)], axis=-1)
    y1 = jnp.maximum(jnp.dot(lhs3, f1_ref[...], preferred_element_type=f32)
                     + bf1_ref[...], 0.0).astype(bf16)
    y2 = jnp.maximum(jnp.dot(y1, w2_ref[...], preferred_element_type=f32)
                     + bf2_ref[...], 0.0).astype(bf16)
    o_ref[...] = (jnp.dot(y2, w3_ref[...], preferred_element_type=f32)
                  + b3_ref[...])


def kernel(x, t1, b1, t2, b2, f1, bf1, w2, bf2, w3, b3):
    n = x.shape[0]
    B = next(b for b in (128, 64, 32, 16, 8, 4, 2, 1) if n % b == 0)

    bf16 = jnp.bfloat16
    xr = x.reshape(n, C_IN * H_IN, W_IN).astype(jnp.float32)
    T1 = t1.reshape(KC1, L1).astype(bf16)
    T2 = t2.reshape(KC2, L2).astype(bf16)
    F1 = f1.reshape(KF1, FC1).astype(bf16)
    W2 = w2.astype(bf16)
    W3 = w3.astype(bf16)

    def full(shape):
        return pl.BlockSpec(shape, lambda i: (0,) * len(shape))

    out = pl.pallas_call(
        _fused_kernel,
        out_shape=jax.ShapeDtypeStruct((n, NCLS), jnp.float32),
        grid=(n // B,),
        in_specs=[
            pl.BlockSpec((B, C_IN * H_IN, W_IN), lambda i: (i, 0, 0)),
            full((KC1, L1)),
            full((1, L1)),
            full((KC2, L2)),
            full((1, L2)),
            full((KF1, FC1)),
            full((1, FC1)),
            full((FC1, FC2)),
            full((1, FC2)),
            full((FC2, NCLS)),
            full((1, NCLS)),
        ],
        out_specs=pl.BlockSpec((B, NCLS), lambda i: (i, 0)),
        compiler_params=pltpu.CompilerParams(
            dimension_semantics=("parallel",)),
    )(xr, T1, b1, T2, b2, F1, bf1, W2, bf2, W3, b3)
    return out


# speedup vs baseline: 14.1104x; 1.4709x over previous
import jax
import jax.numpy as jnp
from jax.experimental import pallas as pl
from jax.experimental.pallas import tpu as pltpu

C_IN, H_IN, W_IN = 3, 32, 32
K = 5
C1, H1, W1 = 6, 28, 28
C2, H2, W2 = 16, 10, 10
PH2, PW2 = 5, 5
FC1, FC2, NCLS = 120, 84, 10

L1 = W1 * C1
L1P = L1 - C1
L2 = W2 * C2
L2P = L2 - C2
KC1 = C_IN * K * W_IN
LKP = 256
KC2 = K * LKP
KF1 = PH2 * L2P
PH1P = 16
PH2P = 8


def _fused_kernel(x_ref, t1_ref, b1_ref, t2_ref, b2_ref, f1_ref, bf1_ref,
                  w2_ref, bf2_ref, w3_ref, b3_ref, o_ref,
                  sx_ref, sm1a_ref, sm1b_ref, sp1_ref, sm2a_ref, sm2b_ref,
                  sp2a_ref, sp2b_ref):
    f32 = jnp.float32
    bf16 = jnp.bfloat16
    B = x_ref.shape[0]
    M1 = B * H_IN
    M2 = B * PH1P
    xb = x_ref[...].astype(bf16)

    xcat = jnp.concatenate(
        [xb[:, ci * H_IN:(ci + 1) * H_IN, :] for ci in range(C_IN)],
        axis=-1)
    sx_ref[0:M1, :] = xcat.reshape(M1, C_IN * W_IN)
    sx_ref[M1:M1 + 8, :] = jnp.zeros((8, C_IN * W_IN), bf16)
    lhs1 = jnp.concatenate(
        [sx_ref[pl.ds(kh, M1), :] for kh in range(K)], axis=-1)
    c1 = jnp.dot(lhs1, t1_ref[...],
                 preferred_element_type=f32) + b1_ref[...]
    c1 = jnp.maximum(c1, 0.0)

    wm1 = jnp.maximum(c1[:, 0:L1P], c1[:, C1:L1])
    sm1a_ref[...] = wm1[:, 0:128]
    sm1b_ref[:, 0:L1P - 128] = wm1[:, 128:L1P]
    p1 = jnp.concatenate(
        [jnp.maximum(sm1a_ref[pl.ds(0, M2, 2), :],
                     sm1a_ref[pl.ds(1, M2, 2), :]),
         jnp.maximum(sm1b_ref[pl.ds(0, M2, 2), 0:L1P - 128],
                     sm1b_ref[pl.ds(1, M2, 2), 0:L1P - 128])],
        axis=-1).astype(bf16)

    sp1_ref[0:M2, :] = jnp.pad(p1, ((0, 0), (0, LKP - L1P)))
    sp1_ref[M2:M2 + 8, :] = jnp.zeros((8, LKP), bf16)
    lhs2 = jnp.concatenate(
        [sp1_ref[pl.ds(kh, M2), :] for kh in range(K)], axis=-1)
    c2 = jnp.dot(lhs2, t2_ref[...],
                 preferred_element_type=f32) + b2_ref[...]
    c2 = jnp.maximum(c2, 0.0)

    wm2 = jnp.maximum(c2[:, 0:L2P], c2[:, C2:L2])
    sm2a_ref[...] = wm2[:, 0:128]
    sm2b_ref[:, 0:L2P - 128] = wm2[:, 128:L2P]
    M3 = B * PH2P
    sp2a_ref[...] = jnp.maximum(sm2a_ref[pl.ds(0, M3, 2), :],
                                sm2a_ref[pl.ds(1, M3, 2), :])
    sp2b_ref[...] = jnp.maximum(sm2b_ref[pl.ds(0, M3, 2), :],
                                sm2b_ref[pl.ds(1, M3, 2), :])

    lhs3 = jnp.concatenate(
        [piece
         for r in range(PH2)
         for piece in (sp2a_ref[pl.ds(r, B, PH2P), :],
                       sp2b_ref[pl.ds(r, B, PH2P), 0:L2P - 128])],
        axis=-1).astype(bf16)
    y1 = jnp.maximum(jnp.dot(lhs3, f1_ref[...], preferred_element_type=f32)
                     + bf1_ref[...], 0.0).astype(bf16)
    y2 = jnp.maximum(jnp.dot(y1, w2_ref[...], preferred_element_type=f32)
                     + bf2_ref[...], 0.0).astype(bf16)
    o_ref[...] = (jnp.dot(y2, w3_ref[...], preferred_element_type=f32)
                  + b3_ref[...])


def kernel(x, t1, b1, t2, b2, f1, bf1, w2, bf2, w3, b3):
    n = x.shape[0]
    B = next(b for b in (128, 64, 32, 16, 8, 4, 2, 1) if n % b == 0)

    bf16 = jnp.bfloat16
    xr = x.reshape(n, C_IN * H_IN, W_IN).astype(jnp.float32)
    T1 = t1.reshape(KC1, L1).astype(bf16)
    T2 = jnp.pad(t2, ((0, 0), (0, LKP - L1P), (0, 0))
                 ).reshape(KC2, L2).astype(bf16)
    F1 = f1.reshape(KF1, FC1).astype(bf16)
    W2 = w2.astype(bf16)
    W3 = w3.astype(bf16)

    def full(shape):
        return pl.BlockSpec(shape, lambda i: (0,) * len(shape))

    out = pl.pallas_call(
        _fused_kernel,
        out_shape=jax.ShapeDtypeStruct((n, NCLS), jnp.float32),
        grid=(n // B,),
        in_specs=[
            pl.BlockSpec((B, C_IN * H_IN, W_IN), lambda i: (i, 0, 0)),
            full((KC1, L1)),
            full((1, L1)),
            full((KC2, L2)),
            full((1, L2)),
            full((KF1, FC1)),
            full((1, FC1)),
            full((FC1, FC2)),
            full((1, FC2)),
            full((FC2, NCLS)),
            full((1, NCLS)),
        ],
        out_specs=pl.BlockSpec((B, NCLS), lambda i: (i, 0)),
        scratch_shapes=[
            pltpu.VMEM((B * H_IN + 8, C_IN * W_IN), bf16),
            pltpu.VMEM((B * H_IN, 128), jnp.float32),
            pltpu.VMEM((B * H_IN, 128), jnp.float32),
            pltpu.VMEM((B * PH1P + 8, LKP), bf16),
            pltpu.VMEM((B * PH1P, 128), jnp.float32),
            pltpu.VMEM((B * PH1P, 128), jnp.float32),
            pltpu.VMEM((B * PH2P, 128), jnp.float32),
            pltpu.VMEM((B * PH2P, 128), jnp.float32),
        ],
        compiler_params=pltpu.CompilerParams(
            dimension_semantics=("parallel",)),
    )(xr, T1, b1, T2, b2, F1, bf1, W2, bf2, W3, b3)
    return out
